# baseline (speedup 1.0000x reference)
import jax
import jax.numpy as jnp
from jax.experimental import pallas as pl
from jax.experimental.pallas import tpu as pltpu

_CROP = 24


def _axis(lo, hi, size, crop):
    scale = (hi - lo) * (size - 1.0) / (crop - 1.0)
    c = lo[:, None] * (size - 1.0) + (
        jnp.arange(crop, dtype=jnp.float32)[None, :] * scale[:, None]
    )
    valid = (c >= 0.0) & (c <= size - 1.0)
    c0 = jnp.floor(c)
    frac = c - c0
    i0 = jnp.clip(c0.astype(jnp.int32), 0, size - 1)
    i1 = jnp.clip(i0 + 1, 0, size - 1)
    return i0, i1, frac, valid


def _weight_matrix(i0, i1, frac, valid, size):
    oh0 = jax.nn.one_hot(i0, size, dtype=jnp.float32)
    oh1 = jax.nn.one_hot(i1, size, dtype=jnp.float32)
    w = oh0 * (1.0 - frac)[..., None] + oh1 * frac[..., None]
    return w * valid.astype(jnp.float32)[..., None]


def _interp_kernel(bi_ref, z0_ref, z1_ref, wz_ref, wy_ref, wxt_ref,
                   p0_ref, p1_ref, out_ref):
    b = pl.program_id(0)
    z = pl.program_id(1)
    p0 = p0_ref[0, :, 0, :, :]
    p1 = p1_ref[0, :, 0, :, :]
    p = (p0 * wz_ref[b, z, 0] + p1 * wz_ref[b, z, 1]).astype(jnp.bfloat16)
    s = jax.lax.dot_general(p.reshape(32 * 64, 64), wxt_ref[0],
                            (((1,), (0,)), ((), ())),
                            preferred_element_type=jnp.float32)
    s4 = (s.reshape(32, 64, 24).astype(jnp.bfloat16)
          .transpose(1, 0, 2).reshape(64, 32 * 24))
    q = jax.lax.dot_general(wy_ref[0], s4, (((1,), (0,)), ((), ())),
                            preferred_element_type=jnp.float32)
    out_ref[0, :, 0] = q.reshape(24, 32, 24).transpose(1, 0, 2)


def kernel(image, boxes, box_ind):
    n, c, d, h, w = image.shape
    bz1, by1, bx1, bz2, by2, bx2 = (boxes[:, i] for i in range(6))
    z0, z1, fz, vz = _axis(bz1, bz2, d, _CROP)
    y0, y1, fy, vy = _axis(by1, by2, h, _CROP)
    x0, x1, fx, vx = _axis(bx1, bx2, w, _CROP)

    vzf = vz.astype(jnp.float32)
    wz = jnp.stack([(1.0 - fz) * vzf, fz * vzf], axis=-1)
    wy = _weight_matrix(y0, y1, fy, vy, h).astype(jnp.bfloat16)
    wxt = jnp.transpose(_weight_matrix(x0, x1, fx, vx, w),
                        (0, 2, 1)).astype(jnp.bfloat16)

    bi = box_ind.astype(jnp.int32)
    nb = boxes.shape[0]
    imgb = image.astype(jnp.bfloat16)

    grid_spec = pltpu.PrefetchScalarGridSpec(
        num_scalar_prefetch=4,
        grid=(nb, _CROP),
        in_specs=[
            pl.BlockSpec((1, _CROP, h), lambda b, z, *sp: (b, 0, 0)),
            pl.BlockSpec((1, w, _CROP), lambda b, z, *sp: (b, 0, 0)),
            pl.BlockSpec((1, c, 1, h, w),
                         lambda b, z, bi_, z0_, z1_, wz_: (bi_[b], 0, z0_[b, z], 0, 0)),
            pl.BlockSpec((1, c, 1, h, w),
                         lambda b, z, bi_, z0_, z1_, wz_: (bi_[b], 0, z1_[b, z], 0, 0)),
        ],
        out_specs=pl.BlockSpec((1, c, 1, _CROP, _CROP),
                               lambda b, z, *sp: (b, 0, z, 0, 0)),
    )

    return pl.pallas_call(
        _interp_kernel,
        grid_spec=grid_spec,
        out_shape=jax.ShapeDtypeStruct((nb, c, _CROP, _CROP, _CROP), jnp.float32),
        compiler_params=pltpu.CompilerParams(
            dimension_semantics=("arbitrary", "arbitrary"),
        ),
    )(bi, z0, z1, wz, wy, wxt, imgb, imgb)


# speedup vs baseline: 1.0803x; 1.0803x over previous
import jax
import jax.numpy as jnp
from jax.experimental import pallas as pl
from jax.experimental.pallas import tpu as pltpu

_CROP = 24
_CB = 8


def _axis(lo, hi, size, crop):
    scale = (hi - lo) * (size - 1.0) / (crop - 1.0)
    c = lo[:, None] * (size - 1.0) + (
        jnp.arange(crop, dtype=jnp.float32)[None, :] * scale[:, None]
    )
    valid = (c >= 0.0) & (c <= size - 1.0)
    c0 = jnp.floor(c)
    frac = c - c0
    i0 = jnp.clip(c0.astype(jnp.int32), 0, size - 1)
    i1 = jnp.clip(i0 + 1, 0, size - 1)
    return i0, i1, frac, valid


def _weight_matrix(i0, i1, frac, valid, size):
    oh0 = jax.nn.one_hot(i0, size, dtype=jnp.float32)
    oh1 = jax.nn.one_hot(i1, size, dtype=jnp.float32)
    w = oh0 * (1.0 - frac)[..., None] + oh1 * frac[..., None]
    return w * valid.astype(jnp.float32)[..., None]


def _interp_kernel(ord_ref, bis_ref, z0_ref, wz_ref, wy_ref, wxt_ref,
                   vol_ref, out_ref):
    b = pl.program_id(1)
    wyb = wy_ref[0]
    wxtb = wxt_ref[0]
    for z in range(_CROP):
        zi = z0_ref[b, z]
        p0 = vol_ref[0, :, zi, :, :]
        p1 = vol_ref[0, :, zi + 1, :, :]
        p = (p0 * wz_ref[b, z, 0] + p1 * wz_ref[b, z, 1]).astype(jnp.bfloat16)
        s = jax.lax.dot_general(p.reshape(_CB * 64, 64), wxtb,
                                (((1,), (0,)), ((), ())),
                                preferred_element_type=jnp.float32)
        s4 = (s.reshape(_CB, 64, _CROP).astype(jnp.bfloat16)
              .transpose(1, 0, 2).reshape(64, _CB * _CROP))
        q = jax.lax.dot_general(wyb, s4, (((1,), (0,)), ((), ())),
                                preferred_element_type=jnp.float32)
        out_ref[0, :, z, :, :] = q.reshape(_CROP, _CB, _CROP).transpose(1, 0, 2)


def kernel(image, boxes, box_ind):
    n, c, d, h, w = image.shape
    bz1, by1, bx1, bz2, by2, bx2 = (boxes[:, i] for i in range(6))
    z0, z1, fz, vz = _axis(bz1, bz2, d, _CROP)
    y0, y1, fy, vy = _axis(by1, by2, h, _CROP)
    x0, x1, fx, vx = _axis(bx1, bx2, w, _CROP)

    z0s = jnp.minimum(z0, d - 2)
    fzs = fz + (z0 - z0s).astype(jnp.float32)
    vzf = vz.astype(jnp.float32)
    wz = jnp.stack([(1.0 - fzs) * vzf, fzs * vzf], axis=-1)
    wy = _weight_matrix(y0, y1, fy, vy, h).astype(jnp.bfloat16)
    wxt = jnp.transpose(_weight_matrix(x0, x1, fx, vx, w),
                        (0, 2, 1)).astype(jnp.bfloat16)

    bi = box_ind.astype(jnp.int32)
    nb = boxes.shape[0]
    order = jnp.argsort(bi).astype(jnp.int32)

    grid_spec = pltpu.PrefetchScalarGridSpec(
        num_scalar_prefetch=4,
        grid=(c // _CB, nb),
        in_specs=[
            pl.BlockSpec((1, _CROP, h), lambda cb, b, *sp: (b, 0, 0)),
            pl.BlockSpec((1, w, _CROP), lambda cb, b, *sp: (b, 0, 0)),
            pl.BlockSpec((1, _CB, d, h, w),
                         lambda cb, b, ord_, bis_, z0_, wz_: (bis_[b], cb, 0, 0, 0)),
        ],
        out_specs=pl.BlockSpec((1, _CB, _CROP, _CROP, _CROP),
                               lambda cb, b, ord_, bis_, z0_, wz_: (ord_[b], cb, 0, 0, 0)),
    )

    return pl.pallas_call(
        _interp_kernel,
        grid_spec=grid_spec,
        out_shape=jax.ShapeDtypeStruct((nb, c, _CROP, _CROP, _CROP), jnp.float32),
        compiler_params=pltpu.CompilerParams(
            dimension_semantics=("arbitrary", "arbitrary"),
        ),
    )(order, bi[order], z0s[order], wz[order], wy[order], wxt[order], image)


# speedup vs baseline: 1.2000x; 1.1108x over previous
import jax
import jax.numpy as jnp
from jax.experimental import pallas as pl
from jax.experimental.pallas import tpu as pltpu

_CROP = 24
_NQ = 4


def _axis(lo, hi, size, crop):
    scale = (hi - lo) * (size - 1.0) / (crop - 1.0)
    c = lo[:, None] * (size - 1.0) + (
        jnp.arange(crop, dtype=jnp.float32)[None, :] * scale[:, None]
    )
    valid = (c >= 0.0) & (c <= size - 1.0)
    c0 = jnp.floor(c)
    frac = c - c0
    i0 = jnp.clip(c0.astype(jnp.int32), 0, size - 1)
    i1 = jnp.clip(i0 + 1, 0, size - 1)
    return i0, i1, frac, valid


def _weight_matrix(i0, i1, frac, valid, size):
    oh0 = jax.nn.one_hot(i0, size, dtype=jnp.float32)
    oh1 = jax.nn.one_hot(i1, size, dtype=jnp.float32)
    w = oh0 * (1.0 - frac)[..., None] + oh1 * frac[..., None]
    return w * valid.astype(jnp.float32)[..., None]


def _interp_kernel(bi_ref, z0_ref, wz_ref, wy_ref, wxt_ref, *refs):
    plane_refs, out_ref = refs[:-1], refs[-1]
    b = pl.program_id(0)
    z = pl.program_id(1)
    w0 = wz_ref[b, z, 0]
    w1 = wz_ref[b, z, 1]
    wyb = wy_ref[0]
    wxtb = wxt_ref[0]
    cq = 32 // _NQ
    for q in range(_NQ):
        p0 = plane_refs[q][0, :, 0, :, :]
        p1 = plane_refs[_NQ + q][0, :, 0, :, :]
        p = (p0 * w0 + p1 * w1).astype(jnp.bfloat16)
        s = jax.lax.dot_general(p.reshape(cq * 64, 64), wxtb,
                                (((1,), (0,)), ((), ())),
                                preferred_element_type=jnp.float32)
        s3 = s.astype(jnp.bfloat16).reshape(cq, 64, _CROP)
        for ci in range(cq):
            qv = jax.lax.dot_general(wyb, s3[ci], (((1,), (0,)), ((), ())),
                                     preferred_element_type=jnp.float32)
            out_ref[0, q * cq + ci, 0, :, :] = qv


def kernel(image, boxes, box_ind):
    n, c, d, h, w = image.shape
    bz1, by1, bx1, bz2, by2, bx2 = (boxes[:, i] for i in range(6))
    z0, z1, fz, vz = _axis(bz1, bz2, d, _CROP)
    y0, y1, fy, vy = _axis(by1, by2, h, _CROP)
    x0, x1, fx, vx = _axis(bx1, bx2, w, _CROP)

    z0s = jnp.minimum(z0, d - 2)
    fzs = fz + (z0 - z0s).astype(jnp.float32)
    vzf = vz.astype(jnp.float32)
    wz = jnp.stack([(1.0 - fzs) * vzf, fzs * vzf], axis=-1)
    wy = _weight_matrix(y0, y1, fy, vy, h).astype(jnp.bfloat16)
    wxt = jnp.transpose(_weight_matrix(x0, x1, fx, vx, w),
                        (0, 2, 1)).astype(jnp.bfloat16)

    bi = box_ind.astype(jnp.int32)
    nb = boxes.shape[0]
    cq = c // _NQ

    def plane_spec(q, zref_sel):
        def imap(b_, z_, bi_, z0_, wz_):
            return (bi_[b_], q, z0_[b_, z_] + zref_sel, 0, 0)
        return pl.BlockSpec((1, cq, 1, h, w), imap)

    in_specs = [
        pl.BlockSpec((1, _CROP, h), lambda b_, z_, *sp: (b_, 0, 0)),
        pl.BlockSpec((1, w, _CROP), lambda b_, z_, *sp: (b_, 0, 0)),
    ]
    in_specs += [plane_spec(q, 0) for q in range(_NQ)]
    in_specs += [plane_spec(q, 1) for q in range(_NQ)]

    grid_spec = pltpu.PrefetchScalarGridSpec(
        num_scalar_prefetch=3,
        grid=(nb, _CROP),
        in_specs=in_specs,
        out_specs=pl.BlockSpec((1, c, 1, _CROP, _CROP),
                               lambda b_, z_, *sp: (b_, 0, z_, 0, 0)),
    )

    return pl.pallas_call(
        _interp_kernel,
        grid_spec=grid_spec,
        out_shape=jax.ShapeDtypeStruct((nb, c, _CROP, _CROP, _CROP), jnp.float32),
        compiler_params=pltpu.CompilerParams(
            dimension_semantics=("arbitrary", "arbitrary"),
        ),
    )(bi, z0s, wz, wy, wxt, *([image] * (2 * _NQ)))


# speedup vs baseline: 1.2931x; 1.0776x over previous
import jax
import jax.numpy as jnp
from jax.experimental import pallas as pl
from jax.experimental.pallas import tpu as pltpu

_CROP = 24
_NQ = 4


def _axis(lo, hi, size, crop):
    scale = (hi - lo) * (size - 1.0) / (crop - 1.0)
    c = lo[:, None] * (size - 1.0) + (
        jnp.arange(crop, dtype=jnp.float32)[None, :] * scale[:, None]
    )
    valid = (c >= 0.0) & (c <= size - 1.0)
    c0 = jnp.floor(c)
    frac = c - c0
    i0 = jnp.clip(c0.astype(jnp.int32), 0, size - 1)
    i1 = jnp.clip(i0 + 1, 0, size - 1)
    return i0, i1, frac, valid


def _weight_matrix(i0, i1, frac, valid, size):
    oh0 = jax.nn.one_hot(i0, size, dtype=jnp.float32)
    oh1 = jax.nn.one_hot(i1, size, dtype=jnp.float32)
    w = oh0 * (1.0 - frac)[..., None] + oh1 * frac[..., None]
    return w * valid.astype(jnp.float32)[..., None]


def _interp_kernel(bi_ref, e2_ref, o2_ref, wz_ref, wy_ref, wxt_ref, *refs):
    plane_refs, out_ref = refs[:-1], refs[-1]
    b = pl.program_id(0)
    z = pl.program_id(1)
    w0 = wz_ref[b, z, 0]
    w1 = wz_ref[b, z, 1]
    wyb = wy_ref[0]
    wxtb = wxt_ref[0]
    cq = 32 // _NQ
    for q in range(_NQ):
        p0 = plane_refs[q][0, :, 0, 0, :, :]
        p1 = plane_refs[_NQ + q][0, :, 0, 0, :, :]
        p = (p0 * w0 + p1 * w1).astype(jnp.bfloat16)
        s = jax.lax.dot_general(p.reshape(cq * 64, 64), wxtb,
                                (((1,), (0,)), ((), ())),
                                preferred_element_type=jnp.float32)
        s3 = s.astype(jnp.bfloat16).reshape(cq, 64, _CROP)
        for ci in range(cq):
            qv = jax.lax.dot_general(wyb, s3[ci], (((1,), (0,)), ((), ())),
                                     preferred_element_type=jnp.float32)
            out_ref[0, q * cq + ci, 0, :, :] = qv


def kernel(image, boxes, box_ind):
    n, c, d, h, w = image.shape
    bz1, by1, bx1, bz2, by2, bx2 = (boxes[:, i] for i in range(6))
    z0, z1, fz, vz = _axis(bz1, bz2, d, _CROP)
    y0, y1, fy, vy = _axis(by1, by2, h, _CROP)
    x0, x1, fx, vx = _axis(bx1, bx2, w, _CROP)

    z0s = jnp.minimum(z0, d - 2)
    fzs = fz + (z0 - z0s).astype(jnp.float32)
    vzf = vz.astype(jnp.float32)
    par = z0s & 1
    e2 = (z0s + par) >> 1
    o2 = (z0s + 1 - par) >> 1
    w_lo = (1.0 - fzs) * vzf
    w_hi = fzs * vzf
    even_is_lo = (par == 0)
    wz = jnp.stack([jnp.where(even_is_lo, w_lo, w_hi),
                    jnp.where(even_is_lo, w_hi, w_lo)], axis=-1)
    wy = _weight_matrix(y0, y1, fy, vy, h).astype(jnp.bfloat16)
    wxt = jnp.transpose(_weight_matrix(x0, x1, fx, vx, w),
                        (0, 2, 1)).astype(jnp.bfloat16)

    bi = box_ind.astype(jnp.int32)
    nb = boxes.shape[0]
    cq = c // _NQ

    img6 = image.reshape(n, c, d // 2, 2, h, w)

    def plane_spec(q, parity):
        def imap(b_, z_, bi_, e2_, o2_, wz_):
            zi = e2_[b_, z_] if parity == 0 else o2_[b_, z_]
            return (bi_[b_], q, zi, parity, 0, 0)
        return pl.BlockSpec((1, cq, 1, 1, h, w), imap)

    in_specs = [
        pl.BlockSpec((1, _CROP, h), lambda b_, z_, *sp: (b_, 0, 0)),
        pl.BlockSpec((1, w, _CROP), lambda b_, z_, *sp: (b_, 0, 0)),
    ]
    in_specs += [plane_spec(q, 0) for q in range(_NQ)]
    in_specs += [plane_spec(q, 1) for q in range(_NQ)]

    grid_spec = pltpu.PrefetchScalarGridSpec(
        num_scalar_prefetch=4,
        grid=(nb, _CROP),
        in_specs=in_specs,
        out_specs=pl.BlockSpec((1, c, 1, _CROP, _CROP),
                               lambda b_, z_, *sp: (b_, 0, z_, 0, 0)),
    )

    return pl.pallas_call(
        _interp_kernel,
        grid_spec=grid_spec,
        out_shape=jax.ShapeDtypeStruct((nb, c, _CROP, _CROP, _CROP), jnp.float32),
        compiler_params=pltpu.CompilerParams(
            dimension_semantics=("arbitrary", "arbitrary"),
        ),
    )(bi, e2, o2, wz, wy, wxt, *([img6] * (2 * _NQ)))


# speedup vs baseline: 1.3142x; 1.0163x over previous
import jax
import jax.numpy as jnp
from jax.experimental import pallas as pl
from jax.experimental.pallas import tpu as pltpu

_CROP = 24
_NQ = 1


def _axis(lo, hi, size, crop):
    scale = (hi - lo) * (size - 1.0) / (crop - 1.0)
    c = lo[:, None] * (size - 1.0) + (
        jnp.arange(crop, dtype=jnp.float32)[None, :] * scale[:, None]
    )
    valid = (c >= 0.0) & (c <= size - 1.0)
    c0 = jnp.floor(c)
    frac = c - c0
    i0 = jnp.clip(c0.astype(jnp.int32), 0, size - 1)
    i1 = jnp.clip(i0 + 1, 0, size - 1)
    return i0, i1, frac, valid


def _weight_matrix(i0, i1, frac, valid, size):
    oh0 = jax.nn.one_hot(i0, size, dtype=jnp.float32)
    oh1 = jax.nn.one_hot(i1, size, dtype=jnp.float32)
    w = oh0 * (1.0 - frac)[..., None] + oh1 * frac[..., None]
    return w * valid.astype(jnp.float32)[..., None]


def _interp_kernel(bi_ref, e2_ref, o2_ref, wz_ref, wy_ref, wxt_ref, *refs):
    plane_refs, out_ref = refs[:-1], refs[-1]
    b = pl.program_id(0)
    z = pl.program_id(1)
    w0 = wz_ref[b, z, 0]
    w1 = wz_ref[b, z, 1]
    wyb = wy_ref[0]
    wxtb = wxt_ref[0]
    cq = 32 // _NQ
    for q in range(_NQ):
        p0 = plane_refs[q][0, :, 0, 0, :, :]
        p1 = plane_refs[_NQ + q][0, :, 0, 0, :, :]
        p = (p0 * w0 + p1 * w1).astype(jnp.bfloat16)
        s = jax.lax.dot_general(p.reshape(cq * 64, 64), wxtb,
                                (((1,), (0,)), ((), ())),
                                preferred_element_type=jnp.float32)
        s3 = s.astype(jnp.bfloat16).reshape(cq, 64, _CROP)
        for ci in range(cq):
            qv = jax.lax.dot_general(wyb, s3[ci], (((1,), (0,)), ((), ())),
                                     preferred_element_type=jnp.float32)
            out_ref[0, q * cq + ci, 0, :, :] = qv


def kernel(image, boxes, box_ind):
    n, c, d, h, w = image.shape
    bz1, by1, bx1, bz2, by2, bx2 = (boxes[:, i] for i in range(6))
    z0, z1, fz, vz = _axis(bz1, bz2, d, _CROP)
    y0, y1, fy, vy = _axis(by1, by2, h, _CROP)
    x0, x1, fx, vx = _axis(bx1, bx2, w, _CROP)

    z0s = jnp.minimum(z0, d - 2)
    fzs = fz + (z0 - z0s).astype(jnp.float32)
    vzf = vz.astype(jnp.float32)
    par = z0s & 1
    e2 = (z0s + par) >> 1
    o2 = (z0s + 1 - par) >> 1
    w_lo = (1.0 - fzs) * vzf
    w_hi = fzs * vzf
    even_is_lo = (par == 0)
    wz = jnp.stack([jnp.where(even_is_lo, w_lo, w_hi),
                    jnp.where(even_is_lo, w_hi, w_lo)], axis=-1)
    wy = _weight_matrix(y0, y1, fy, vy, h).astype(jnp.bfloat16)
    wxt = jnp.transpose(_weight_matrix(x0, x1, fx, vx, w),
                        (0, 2, 1)).astype(jnp.bfloat16)

    bi = box_ind.astype(jnp.int32)
    nb = boxes.shape[0]
    cq = c // _NQ

    img6 = image.reshape(n, c, d // 2, 2, h, w)

    def plane_spec(q, parity):
        def imap(b_, z_, bi_, e2_, o2_, wz_):
            zi = e2_[b_, z_] if parity == 0 else o2_[b_, z_]
            return (bi_[b_], q, zi, parity, 0, 0)
        return pl.BlockSpec((1, cq, 1, 1, h, w), imap)

    in_specs = [
        pl.BlockSpec((1, _CROP, h), lambda b_, z_, *sp: (b_, 0, 0)),
        pl.BlockSpec((1, w, _CROP), lambda b_, z_, *sp: (b_, 0, 0)),
    ]
    in_specs += [plane_spec(q, 0) for q in range(_NQ)]
    in_specs += [plane_spec(q, 1) for q in range(_NQ)]

    grid_spec = pltpu.PrefetchScalarGridSpec(
        num_scalar_prefetch=4,
        grid=(nb, _CROP),
        in_specs=in_specs,
        out_specs=pl.BlockSpec((1, c, 1, _CROP, _CROP),
                               lambda b_, z_, *sp: (b_, 0, z_, 0, 0)),
    )

    return pl.pallas_call(
        _interp_kernel,
        grid_spec=grid_spec,
        out_shape=jax.ShapeDtypeStruct((nb, c, _CROP, _CROP, _CROP), jnp.float32),
        compiler_params=pltpu.CompilerParams(
            dimension_semantics=("arbitrary", "arbitrary"),
        ),
    )(bi, e2, o2, wz, wy, wxt, *([img6] * (2 * _NQ)))


# speedup vs baseline: 1.3189x; 1.0036x over previous
import jax
import jax.numpy as jnp
from jax.experimental import pallas as pl
from jax.experimental.pallas import tpu as pltpu

_CROP = 24
_NQ = 2


def _axis(lo, hi, size, crop):
    scale = (hi - lo) * (size - 1.0) / (crop - 1.0)
    c = lo[:, None] * (size - 1.0) + (
        jnp.arange(crop, dtype=jnp.float32)[None, :] * scale[:, None]
    )
    valid = (c >= 0.0) & (c <= size - 1.0)
    c0 = jnp.floor(c)
    frac = c - c0
    i0 = jnp.clip(c0.astype(jnp.int32), 0, size - 1)
    i1 = jnp.clip(i0 + 1, 0, size - 1)
    return i0, i1, frac, valid


def _weight_matrix(i0, i1, frac, valid, size):
    oh0 = jax.nn.one_hot(i0, size, dtype=jnp.float32)
    oh1 = jax.nn.one_hot(i1, size, dtype=jnp.float32)
    w = oh0 * (1.0 - frac)[..., None] + oh1 * frac[..., None]
    return w * valid.astype(jnp.float32)[..., None]


def _interp_kernel(bi_ref, e2_ref, o2_ref, wz_ref, wy_ref, wxt_ref, *refs):
    plane_refs, out_ref = refs[:-1], refs[-1]
    b = pl.program_id(0)
    z = pl.program_id(1)
    w0 = wz_ref[b, z, 0]
    w1 = wz_ref[b, z, 1]
    wyb = wy_ref[0]
    wxtb = wxt_ref[0]
    cq = 32 // _NQ
    for q in range(_NQ):
        p0 = plane_refs[q][0, :, 0, 0, :, :]
        p1 = plane_refs[_NQ + q][0, :, 0, 0, :, :]
        p = (p0 * w0 + p1 * w1).astype(jnp.bfloat16)
        s = jax.lax.dot_general(p.reshape(cq * 64, 64), wxtb,
                                (((1,), (0,)), ((), ())),
                                preferred_element_type=jnp.float32)
        s3 = s.astype(jnp.bfloat16).reshape(cq, 64, _CROP)
        for ci in range(cq):
            qv = jax.lax.dot_general(wyb, s3[ci], (((1,), (0,)), ((), ())),
                                     preferred_element_type=jnp.float32)
            out_ref[0, q * cq + ci, 0, :, :] = qv


def kernel(image, boxes, box_ind):
    n, c, d, h, w = image.shape
    bz1, by1, bx1, bz2, by2, bx2 = (boxes[:, i] for i in range(6))
    z0, z1, fz, vz = _axis(bz1, bz2, d, _CROP)
    y0, y1, fy, vy = _axis(by1, by2, h, _CROP)
    x0, x1, fx, vx = _axis(bx1, bx2, w, _CROP)

    z0s = jnp.minimum(z0, d - 2)
    fzs = fz + (z0 - z0s).astype(jnp.float32)
    vzf = vz.astype(jnp.float32)
    par = z0s & 1
    e2 = (z0s + par) >> 1
    o2 = (z0s + 1 - par) >> 1
    w_lo = (1.0 - fzs) * vzf
    w_hi = fzs * vzf
    even_is_lo = (par == 0)
    wz = jnp.stack([jnp.where(even_is_lo, w_lo, w_hi),
                    jnp.where(even_is_lo, w_hi, w_lo)], axis=-1)
    wy = _weight_matrix(y0, y1, fy, vy, h).astype(jnp.bfloat16)
    wxt = jnp.transpose(_weight_matrix(x0, x1, fx, vx, w),
                        (0, 2, 1)).astype(jnp.bfloat16)

    bi = box_ind.astype(jnp.int32)
    nb = boxes.shape[0]
    cq = c // _NQ

    img6 = image.reshape(n, c, d // 2, 2, h, w)

    def plane_spec(q, parity):
        def imap(b_, z_, bi_, e2_, o2_, wz_):
            zi = e2_[b_, z_] if parity == 0 else o2_[b_, z_]
            return (bi_[b_], q, zi, parity, 0, 0)
        return pl.BlockSpec((1, cq, 1, 1, h, w), imap)

    in_specs = [
        pl.BlockSpec((1, _CROP, h), lambda b_, z_, *sp: (b_, 0, 0)),
        pl.BlockSpec((1, w, _CROP), lambda b_, z_, *sp: (b_, 0, 0)),
    ]
    in_specs += [plane_spec(q, 0) for q in range(_NQ)]
    in_specs += [plane_spec(q, 1) for q in range(_NQ)]

    grid_spec = pltpu.PrefetchScalarGridSpec(
        num_scalar_prefetch=4,
        grid=(nb, _CROP),
        in_specs=in_specs,
        out_specs=pl.BlockSpec((1, c, 1, _CROP, _CROP),
                               lambda b_, z_, *sp: (b_, 0, z_, 0, 0)),
    )

    return pl.pallas_call(
        _interp_kernel,
        grid_spec=grid_spec,
        out_shape=jax.ShapeDtypeStruct((nb, c, _CROP, _CROP, _CROP), jnp.float32),
        compiler_params=pltpu.CompilerParams(
            dimension_semantics=("arbitrary", "arbitrary"),
        ),
    )(bi, e2, o2, wz, wy, wxt, *([img6] * (2 * _NQ)))


# speedup vs baseline: 1.9813x; 1.5022x over previous
import jax
import jax.numpy as jnp
from jax.experimental import pallas as pl
from jax.experimental.pallas import tpu as pltpu

_CROP = 24
_CB = 8


def _axis(lo, hi, size, crop):
    scale = (hi - lo) * (size - 1.0) / (crop - 1.0)
    c = lo[:, None] * (size - 1.0) + (
        jnp.arange(crop, dtype=jnp.float32)[None, :] * scale[:, None]
    )
    valid = (c >= 0.0) & (c <= size - 1.0)
    c0 = jnp.floor(c)
    frac = c - c0
    i0 = jnp.clip(c0.astype(jnp.int32), 0, size - 1)
    i1 = jnp.clip(i0 + 1, 0, size - 1)
    return i0, i1, frac, valid


def _weight_matrix(i0, i1, frac, valid, size):
    oh0 = jax.nn.one_hot(i0, size, dtype=jnp.float32)
    oh1 = jax.nn.one_hot(i1, size, dtype=jnp.float32)
    w = oh0 * (1.0 - frac)[..., None] + oh1 * frac[..., None]
    return w * valid.astype(jnp.float32)[..., None]


def _interp_kernel(ord_ref, bis_ref, z0_ref, wz_ref, wy_ref, wxt_ref,
                   vol_ref, out_ref):
    b = pl.program_id(1)
    wyb = wy_ref[0]
    wxtb = wxt_ref[0]
    for z in range(_CROP):
        zi = z0_ref[b, z]
        p0 = vol_ref[0, :, zi, :, :]
        p1 = vol_ref[0, :, zi + 1, :, :]
        p = (p0 * wz_ref[b, z, 0] + p1 * wz_ref[b, z, 1]).astype(jnp.bfloat16)
        s = jax.lax.dot_general(p.reshape(_CB * 64, 64), wxtb,
                                (((1,), (0,)), ((), ())),
                                preferred_element_type=jnp.float32)
        s3 = s.astype(jnp.bfloat16).reshape(_CB, 64, _CROP)
        for ci in range(_CB):
            qv = jax.lax.dot_general(wyb, s3[ci], (((1,), (0,)), ((), ())),
                                     preferred_element_type=jnp.float32)
            out_ref[0, ci, z, :, :] = qv


def kernel(image, boxes, box_ind):
    n, c, d, h, w = image.shape
    bz1, by1, bx1, bz2, by2, bx2 = (boxes[:, i] for i in range(6))
    z0, z1, fz, vz = _axis(bz1, bz2, d, _CROP)
    y0, y1, fy, vy = _axis(by1, by2, h, _CROP)
    x0, x1, fx, vx = _axis(bx1, bx2, w, _CROP)

    z0s = jnp.minimum(z0, d - 2)
    fzs = fz + (z0 - z0s).astype(jnp.float32)
    vzf = vz.astype(jnp.float32)
    wz = jnp.stack([(1.0 - fzs) * vzf, fzs * vzf], axis=-1)
    wy = _weight_matrix(y0, y1, fy, vy, h).astype(jnp.bfloat16)
    wxt = jnp.transpose(_weight_matrix(x0, x1, fx, vx, w),
                        (0, 2, 1)).astype(jnp.bfloat16)

    bi = box_ind.astype(jnp.int32)
    nb = boxes.shape[0]
    order = jnp.argsort(bi).astype(jnp.int32)

    grid_spec = pltpu.PrefetchScalarGridSpec(
        num_scalar_prefetch=4,
        grid=(c // _CB, nb),
        in_specs=[
            pl.BlockSpec((1, _CROP, h), lambda cb, b, *sp: (b, 0, 0)),
            pl.BlockSpec((1, w, _CROP), lambda cb, b, *sp: (b, 0, 0)),
            pl.BlockSpec((1, _CB, d, h, w),
                         lambda cb, b, ord_, bis_, z0_, wz_: (bis_[b], cb, 0, 0, 0)),
        ],
        out_specs=pl.BlockSpec((1, _CB, _CROP, _CROP, _CROP),
                               lambda cb, b, ord_, bis_, z0_, wz_: (ord_[b], cb, 0, 0, 0)),
    )

    return pl.pallas_call(
        _interp_kernel,
        grid_spec=grid_spec,
        out_shape=jax.ShapeDtypeStruct((nb, c, _CROP, _CROP, _CROP), jnp.float32),
        compiler_params=pltpu.CompilerParams(
            dimension_semantics=("arbitrary", "arbitrary"),
        ),
    )(order, bi[order], z0s[order], wz[order], wy[order], wxt[order], image)
